# baseline (speedup 1.0000x reference)
import functools

import jax
import jax.numpy as jnp
from jax import lax
from jax.experimental import pallas as pl
from jax.experimental.pallas import tpu as pltpu
from jax.experimental.pallas import tpu_sc as plsc

H = 2048
W = 2048
N = 1048576
F = 15
HOPS = 21
PIX = H * W

NW = 32
NP = 1081344
CHUNK = NP // NW
SUB = 4224
NSUB = CHUNK // SUB
NV = SUB // 16

PCHUNK = PIX // NW
PSUB = 4096
PNSUB = PCHUNK // PSUB
PNV = PSUB // 16

_mesh = plsc.VectorSubcoreMesh(core_axis_name="c", subcore_axis_name="s")


_WB = 8192


def _score_body(attr_ref, diff_ref, mean_ref, icov_ref, out_ref):
    a = attr_ref[...]
    m = mean_ref[...]
    c = jnp.maximum(icov_ref[...], 0.0)
    eps = 1e-10
    t5 = a[0:5, :] - m[0:5]
    l9 = jnp.log(jnp.abs(a[6:15, :]) + eps) - m[5:14]
    lsh = jnp.sqrt(a[7:8, :]) / (jnp.sqrt(a[6:7, :]) + eps) - m[14:15]
    cosa = jnp.cos(a[5:6, :]) - m[15:16]
    sina = jnp.sin(a[5:6, :]) - m[16:17]
    s = (jnp.sum(t5 * t5 * c[0:5], axis=0, keepdims=True)
         + jnp.sum(l9 * l9 * c[5:14], axis=0, keepdims=True)
         + lsh * lsh * c[14:15] + cosa * cosa * c[15:16]
         + sina * sina * c[16:17])
    out_ref[...] = (jnp.exp(-s) * diff_ref[0])[None]


def _scores(attributes, maxtree_diff, mean, inv_diagonal_cov):
    grid = N // _WB
    out = pl.pallas_call(
        _score_body,
        grid=(grid,),
        in_specs=[
            pl.BlockSpec((F, _WB), lambda i: (0, i)),
            pl.BlockSpec((1, 1, _WB), lambda i: (i, 0, 0)),
            pl.BlockSpec((17, 1), lambda i: (0, 0)),
            pl.BlockSpec((17, 1), lambda i: (0, 0)),
        ],
        out_specs=pl.BlockSpec((1, 1, _WB), lambda i: (i, 0, 0)),
        out_shape=jax.ShapeDtypeStruct((grid, 1, _WB), jnp.float32),
    )(attributes.T, maxtree_diff.reshape(grid, 1, _WB), mean.reshape(17, 1),
      inv_diagonal_cov.reshape(17, 1))
    return out.reshape(N)


S_MIR = 32768


def _lane_or(v):
    m = v[0]
    for _l in range(1, 16):
        m = m | v[_l]
    return m


def _hop_body(par_in, val_in, par_out, val_out,
              idx_v, idx2_v, val_v, npar_v, gval_v, npar2_v, gval2_v,
              pmir, vmir, sem1, sem2):
    wid = lax.axis_index("s") * 2 + lax.axis_index("c")
    base = wid * CHUNK

    pltpu.sync_copy(par_in.at[pl.ds(0, S_MIR)], pmir)
    pltpu.sync_copy(val_in.at[pl.ds(0, S_MIR)], vmir)

    def sub(j, carry):
        off = base + j * SUB
        pltpu.sync_copy(par_in.at[pl.ds(off, SUB)], idx_v)
        pltpu.sync_copy(val_in.at[pl.ds(off, SUB)], val_v)

        def vor(k, accs):
            accm, acch = accs
            iv = idx_v[pl.ds(k * 16, 16)]
            return accm | iv, acch | lax.shift_right_logical(iv, 15)

        z16 = jnp.zeros((16,), jnp.int32)
        accm, acch = lax.fori_loop(0, NV, vor, (z16, z16))
        m = _lane_or(accm)
        hi = _lane_or(acch)

        @pl.when(m > 0)
        def _live():
            iota16 = lax.iota(jnp.int32, 16)

            @pl.when(hi > 0)
            def _far():
                def fix(k, c):
                    sl = pl.ds(k * 16, 16)
                    own = iota16 + (off + k * 16)
                    iv = idx_v[sl]
                    idx2_v[sl] = jnp.where(iv < S_MIR, own, iv)
                    return c

                lax.fori_loop(0, NV, fix, 0)
                g1 = pltpu.async_copy(par_in.at[idx2_v], npar2_v, sem1)
                g2 = pltpu.async_copy(val_in.at[idx2_v], gval2_v, sem2)
                g1.wait()
                g2.wait()

            def merge(k, c):
                sl = pl.ds(k * 16, 16)
                iv = idx_v[sl]
                mlo = iv < S_MIR
                ivc = iv & (S_MIR - 1)
                gpv = plsc.load_gather(pmir, [ivc])
                glv = plsc.load_gather(vmir, [ivc])
                npar_v[sl] = jnp.where(mlo, gpv, npar2_v[sl])
                val_v[sl] = val_v[sl] + jnp.where(mlo, glv, gval2_v[sl])
                return c

            lax.fori_loop(0, NV, merge, 0)
            pltpu.sync_copy(npar_v, par_out.at[pl.ds(off, SUB)])
            pltpu.sync_copy(val_v, val_out.at[pl.ds(off, SUB)])

        @pl.when(m == 0)
        def _conv():
            pltpu.sync_copy(idx_v, par_out.at[pl.ds(off, SUB)])
            pltpu.sync_copy(val_v, val_out.at[pl.ds(off, SUB)])

        return carry

    lax.fori_loop(0, NSUB, sub, 0)


_hop = functools.partial(
    pl.kernel,
    out_type=(jax.ShapeDtypeStruct((NP,), jnp.int32),
              jax.ShapeDtypeStruct((NP,), jnp.float32)),
    mesh=_mesh,
    compiler_params=pltpu.CompilerParams(needs_layout_passes=False),
    scratch_types=[
        pltpu.VMEM((SUB,), jnp.int32),
        pltpu.VMEM((SUB,), jnp.int32),
        pltpu.VMEM((SUB,), jnp.float32),
        pltpu.VMEM((SUB,), jnp.int32),
        pltpu.VMEM((SUB,), jnp.float32),
        pltpu.VMEM((SUB,), jnp.int32),
        pltpu.VMEM((SUB,), jnp.float32),
        pltpu.VMEM((S_MIR,), jnp.int32),
        pltpu.VMEM((S_MIR,), jnp.float32),
        pltpu.SemaphoreType.DMA,
        pltpu.SemaphoreType.DMA,
    ],
)(_hop_body)


H_SPLIT = 4
NTAIL = HOPS - H_SPLIT
CHUNK16 = NP // 16
NSUB16 = CHUNK16 // SUB

_mesh1 = plsc.VectorSubcoreMesh(core_axis_name="c", subcore_axis_name="s",
                                num_cores=1)


def _tail_body(par_in, val_in, parA, valA, parB, valB,
               idx_v, idx2_v, val_v, npar_v, gval_v, npar2_v, gval2_v,
               pmir, vmir, flags, sem1, sem2):
    tid = lax.axis_index("s")
    base = tid * CHUNK16

    def clear(j, c):
        flags[j] = 0
        return c

    lax.fori_loop(0, NSUB16, clear, 0)

    def one_hop(ps, vs, pd, vd, pd2=None, vd2=None):
        def nlv(j, acc):
            return acc + jnp.where(flags[j] < 1, 1, 0)

        nlive = lax.fori_loop(0, NSUB16, nlv, 0)

        @pl.when(nlive > 0)
        def _mir():
            pltpu.sync_copy(ps.at[pl.ds(0, S_MIR)], pmir)
            pltpu.sync_copy(vs.at[pl.ds(0, S_MIR)], vmir)

        def sub(j, carry):
            f = flags[j]

            @pl.when(f < 1)
            def _do():
                off = base + j * SUB
                pltpu.sync_copy(ps.at[pl.ds(off, SUB)], idx_v)
                pltpu.sync_copy(vs.at[pl.ds(off, SUB)], val_v)

                def vor(k, accs):
                    accm, acch = accs
                    iv = idx_v[pl.ds(k * 16, 16)]
                    return accm | iv, acch | lax.shift_right_logical(iv, 15)

                z16 = jnp.zeros((16,), jnp.int32)
                accm, acch = lax.fori_loop(0, NV, vor, (z16, z16))
                m = _lane_or(accm)
                hi = _lane_or(acch)

                @pl.when(m > 0)
                def _live():
                    iota16 = lax.iota(jnp.int32, 16)

                    @pl.when(hi > 0)
                    def _far():
                        def fix(k, c):
                            sl = pl.ds(k * 16, 16)
                            own = iota16 + (off + k * 16)
                            iv = idx_v[sl]
                            idx2_v[sl] = jnp.where(iv < S_MIR, own, iv)
                            return c

                        lax.fori_loop(0, NV, fix, 0)
                        g1 = pltpu.async_copy(ps.at[idx2_v], npar2_v, sem1)
                        g2 = pltpu.async_copy(vs.at[idx2_v], gval2_v, sem2)
                        g1.wait()
                        g2.wait()

                    def merge(k, c):
                        sl = pl.ds(k * 16, 16)
                        iv = idx_v[sl]
                        mlo = iv < S_MIR
                        ivc = iv & (S_MIR - 1)
                        gpv = plsc.load_gather(pmir, [ivc])
                        glv = plsc.load_gather(vmir, [ivc])
                        npar_v[sl] = jnp.where(mlo, gpv, npar2_v[sl])
                        val_v[sl] = val_v[sl] + jnp.where(mlo, glv, gval2_v[sl])
                        return c

                    lax.fori_loop(0, NV, merge, 0)
                    pltpu.sync_copy(npar_v, pd.at[pl.ds(off, SUB)])
                    pltpu.sync_copy(val_v, vd.at[pl.ds(off, SUB)])

                @pl.when(m == 0)
                def _conv():
                    pltpu.sync_copy(idx_v, pd.at[pl.ds(off, SUB)])
                    pltpu.sync_copy(val_v, vd.at[pl.ds(off, SUB)])
                    if pd2 is not None:
                        pltpu.sync_copy(idx_v, pd2.at[pl.ds(off, SUB)])
                        pltpu.sync_copy(val_v, vd2.at[pl.ds(off, SUB)])
                    flags[j] = 1

            return carry

        lax.fori_loop(0, NSUB16, sub, 0)
        plsc.subcore_barrier()

    def hop(h, c):
        @pl.when(h == 0)
        def _h0():
            one_hop(par_in, val_in, parA, valA, parB, valB)

        @pl.when((h > 0) & (h % 2 == 1))
        def _hodd():
            one_hop(parA, valA, parB, valB)

        @pl.when((h > 0) & (h % 2 == 0))
        def _heven():
            one_hop(parB, valB, parA, valA)

        return c

    lax.fori_loop(0, NTAIL, hop, 0)


_tail = functools.partial(
    pl.kernel,
    out_type=(jax.ShapeDtypeStruct((NP,), jnp.int32),
              jax.ShapeDtypeStruct((NP,), jnp.float32),
              jax.ShapeDtypeStruct((NP,), jnp.int32),
              jax.ShapeDtypeStruct((NP,), jnp.float32)),
    mesh=_mesh1,
    compiler_params=pltpu.CompilerParams(needs_layout_passes=False),
    scratch_types=[
        pltpu.VMEM((SUB,), jnp.int32),
        pltpu.VMEM((SUB,), jnp.int32),
        pltpu.VMEM((SUB,), jnp.float32),
        pltpu.VMEM((SUB,), jnp.int32),
        pltpu.VMEM((SUB,), jnp.float32),
        pltpu.VMEM((SUB,), jnp.int32),
        pltpu.VMEM((SUB,), jnp.float32),
        pltpu.VMEM((S_MIR,), jnp.int32),
        pltpu.VMEM((S_MIR,), jnp.float32),
        pltpu.SMEM((NSUB16,), jnp.int32),
        pltpu.SemaphoreType.DMA,
        pltpu.SemaphoreType.DMA,
    ],
)(_tail_body)


def _pix_body(pix_in, val_in, out_hbm, idxA, idxB, gvA, gvB,
              sgA, sgB, swA, swB):
    wid = lax.axis_index("s") * 2 + lax.axis_index("c")
    base = wid * PCHUNK

    def pair(p, carry):
        offA = base + (2 * p) * PSUB
        offB = offA + PSUB
        pltpu.sync_copy(pix_in.at[pl.ds(offA, PSUB)], idxA)
        cA = pltpu.async_copy(val_in.at[idxA], gvA, sgA)
        pltpu.sync_copy(pix_in.at[pl.ds(offB, PSUB)], idxB)
        cA.wait()
        wA = pltpu.async_copy(gvA, out_hbm.at[pl.ds(offA, PSUB)], swA)
        cB = pltpu.async_copy(val_in.at[idxB], gvB, sgB)
        cB.wait()
        wB = pltpu.async_copy(gvB, out_hbm.at[pl.ds(offB, PSUB)], swB)
        wA.wait()
        wB.wait()
        return carry

    lax.fori_loop(0, PNSUB // 2, pair, 0)


_pix_gather = functools.partial(
    pl.kernel,
    out_type=jax.ShapeDtypeStruct((PIX,), jnp.float32),
    mesh=_mesh,
    scratch_types=[
        pltpu.VMEM((PSUB,), jnp.int32),
        pltpu.VMEM((PSUB,), jnp.int32),
        pltpu.VMEM((PSUB,), jnp.float32),
        pltpu.VMEM((PSUB,), jnp.float32),
        pltpu.SemaphoreType.DMA,
        pltpu.SemaphoreType.DMA,
        pltpu.SemaphoreType.DMA,
        pltpu.SemaphoreType.DMA,
    ],
)(_pix_body)


def kernel(maxtree_parent, pixel_map, maxtree_diff, attributes, mean,
           inv_diagonal_cov):
    contrib = _scores(attributes, maxtree_diff, mean, inv_diagonal_cov)

    par = jnp.zeros((NP,), jnp.int32).at[1:N + 1].set(
        maxtree_parent.astype(jnp.int32) + 1)
    val = jnp.zeros((NP,), jnp.float32).at[1:N + 1].set(contrib)

    for _ in range(H_SPLIT):
        par, val = _hop(par, val)
    _pa, _va, par, val = _tail(par, val)

    pix = (pixel_map + 1).astype(jnp.int32)
    out = _pix_gather(pix, val)
    return out.reshape(H, W)


# speedup vs baseline: 1.0241x; 1.0241x over previous
import functools

import jax
import jax.numpy as jnp
from jax import lax
from jax.experimental import pallas as pl
from jax.experimental.pallas import tpu as pltpu
from jax.experimental.pallas import tpu_sc as plsc

H = 2048
W = 2048
N = 1048576
F = 15
HOPS = 21
PIX = H * W

NW = 32
NP = 1081344
CHUNK = NP // NW
SUB = 4224
NSUB = CHUNK // SUB
NV = SUB // 16

PCHUNK = PIX // NW
PSUB = 4096
PNSUB = PCHUNK // PSUB
PNV = PSUB // 16

_mesh = plsc.VectorSubcoreMesh(core_axis_name="c", subcore_axis_name="s")


_WB = 8192


def _score_body(attr_ref, diff_ref, mean_ref, icov_ref, out_ref):
    a = attr_ref[...]
    m = mean_ref[...]
    c = jnp.maximum(icov_ref[...], 0.0)
    eps = 1e-10
    t5 = a[0:5, :] - m[0:5]
    l9 = jnp.log(jnp.abs(a[6:15, :]) + eps) - m[5:14]
    lsh = jnp.sqrt(a[7:8, :]) / (jnp.sqrt(a[6:7, :]) + eps) - m[14:15]
    cosa = jnp.cos(a[5:6, :]) - m[15:16]
    sina = jnp.sin(a[5:6, :]) - m[16:17]
    s = (jnp.sum(t5 * t5 * c[0:5], axis=0, keepdims=True)
         + jnp.sum(l9 * l9 * c[5:14], axis=0, keepdims=True)
         + lsh * lsh * c[14:15] + cosa * cosa * c[15:16]
         + sina * sina * c[16:17])
    out_ref[...] = (jnp.exp(-s) * diff_ref[0])[None]


def _scores(attributes, maxtree_diff, mean, inv_diagonal_cov):
    grid = N // _WB
    out = pl.pallas_call(
        _score_body,
        grid=(grid,),
        in_specs=[
            pl.BlockSpec((F, _WB), lambda i: (0, i)),
            pl.BlockSpec((1, 1, _WB), lambda i: (i, 0, 0)),
            pl.BlockSpec((17, 1), lambda i: (0, 0)),
            pl.BlockSpec((17, 1), lambda i: (0, 0)),
        ],
        out_specs=pl.BlockSpec((1, 1, _WB), lambda i: (i, 0, 0)),
        out_shape=jax.ShapeDtypeStruct((grid, 1, _WB), jnp.float32),
    )(attributes.T, maxtree_diff.reshape(grid, 1, _WB), mean.reshape(17, 1),
      inv_diagonal_cov.reshape(17, 1))
    return out.reshape(N)


S_MIR = 32768


def _lane_or(v):
    m = v[0]
    for _l in range(1, 16):
        m = m | v[_l]
    return m


def _hop_body(par_in, val_in, par_out, val_out,
              idx_v, idx2_v, val_v, npar_v, gval_v, npar2_v, gval2_v,
              pmir, vmir, sem1, sem2):
    wid = lax.axis_index("s") * 2 + lax.axis_index("c")
    base = wid * CHUNK

    pltpu.sync_copy(par_in.at[pl.ds(0, S_MIR)], pmir)
    pltpu.sync_copy(val_in.at[pl.ds(0, S_MIR)], vmir)

    def sub(j, carry):
        off = base + j * SUB
        pltpu.sync_copy(par_in.at[pl.ds(off, SUB)], idx_v)
        pltpu.sync_copy(val_in.at[pl.ds(off, SUB)], val_v)

        def vor(k, accs):
            accm, acch = accs
            iv = idx_v[pl.ds(k * 16, 16)]
            return accm | iv, acch | lax.shift_right_logical(iv, 15)

        z16 = jnp.zeros((16,), jnp.int32)
        accm, acch = lax.fori_loop(0, NV, vor, (z16, z16))
        m = _lane_or(accm)
        hi = _lane_or(acch)

        @pl.when(m > 0)
        def _live():
            iota16 = lax.iota(jnp.int32, 16)

            @pl.when(hi > 0)
            def _far():
                def fix(k, c):
                    sl = pl.ds(k * 16, 16)
                    own = iota16 + (off + k * 16)
                    iv = idx_v[sl]
                    idx2_v[sl] = jnp.where(iv < S_MIR, own, iv)
                    return c

                lax.fori_loop(0, NV, fix, 0)
                g1 = pltpu.async_copy(par_in.at[idx2_v], npar2_v, sem1)
                g2 = pltpu.async_copy(val_in.at[idx2_v], gval2_v, sem2)
                g1.wait()
                g2.wait()

            def merge(k, c):
                sl = pl.ds(k * 16, 16)
                iv = idx_v[sl]
                mlo = iv < S_MIR
                ivc = iv & (S_MIR - 1)
                gpv = plsc.load_gather(pmir, [ivc])
                glv = plsc.load_gather(vmir, [ivc])
                npar_v[sl] = jnp.where(mlo, gpv, npar2_v[sl])
                val_v[sl] = val_v[sl] + jnp.where(mlo, glv, gval2_v[sl])
                return c

            lax.fori_loop(0, NV, merge, 0)
            pltpu.sync_copy(npar_v, par_out.at[pl.ds(off, SUB)])
            pltpu.sync_copy(val_v, val_out.at[pl.ds(off, SUB)])

        @pl.when(m == 0)
        def _conv():
            pltpu.sync_copy(idx_v, par_out.at[pl.ds(off, SUB)])
            pltpu.sync_copy(val_v, val_out.at[pl.ds(off, SUB)])

        return carry

    lax.fori_loop(0, NSUB, sub, 0)


_hop = functools.partial(
    pl.kernel,
    out_type=(jax.ShapeDtypeStruct((NP,), jnp.int32),
              jax.ShapeDtypeStruct((NP,), jnp.float32)),
    mesh=_mesh,
    compiler_params=pltpu.CompilerParams(needs_layout_passes=False),
    scratch_types=[
        pltpu.VMEM((SUB,), jnp.int32),
        pltpu.VMEM((SUB,), jnp.int32),
        pltpu.VMEM((SUB,), jnp.float32),
        pltpu.VMEM((SUB,), jnp.int32),
        pltpu.VMEM((SUB,), jnp.float32),
        pltpu.VMEM((SUB,), jnp.int32),
        pltpu.VMEM((SUB,), jnp.float32),
        pltpu.VMEM((S_MIR,), jnp.int32),
        pltpu.VMEM((S_MIR,), jnp.float32),
        pltpu.SemaphoreType.DMA,
        pltpu.SemaphoreType.DMA,
    ],
)(_hop_body)


H_SPLIT = 5
NTAIL = HOPS - H_SPLIT
CHUNK16 = NP // 16
NSUB16 = CHUNK16 // SUB

_mesh1 = plsc.VectorSubcoreMesh(core_axis_name="c", subcore_axis_name="s",
                                num_cores=1)


def _tail_body(par_in, val_in, parA, valA, parB, valB,
               idx_v, idx2_v, val_v, npar_v, gval_v, npar2_v, gval2_v,
               pmir, vmir, flags, sem1, sem2):
    tid = lax.axis_index("s")
    base = tid * CHUNK16

    def clear(j, c):
        flags[j] = 0
        return c

    lax.fori_loop(0, NSUB16, clear, 0)

    def one_hop(ps, vs, pd, vd, pd2=None, vd2=None):
        def nlv(j, acc):
            return acc + jnp.where(flags[j] < 1, 1, 0)

        nlive = lax.fori_loop(0, NSUB16, nlv, 0)

        @pl.when(nlive > 0)
        def _mir():
            pltpu.sync_copy(ps.at[pl.ds(0, S_MIR)], pmir)
            pltpu.sync_copy(vs.at[pl.ds(0, S_MIR)], vmir)

        def sub(j, carry):
            f = flags[j]

            @pl.when(f < 1)
            def _do():
                off = base + j * SUB
                pltpu.sync_copy(ps.at[pl.ds(off, SUB)], idx_v)
                pltpu.sync_copy(vs.at[pl.ds(off, SUB)], val_v)

                def vor(k, accs):
                    accm, acch = accs
                    iv = idx_v[pl.ds(k * 16, 16)]
                    return accm | iv, acch | lax.shift_right_logical(iv, 15)

                z16 = jnp.zeros((16,), jnp.int32)
                accm, acch = lax.fori_loop(0, NV, vor, (z16, z16))
                m = _lane_or(accm)
                hi = _lane_or(acch)

                @pl.when(m > 0)
                def _live():
                    iota16 = lax.iota(jnp.int32, 16)

                    @pl.when(hi > 0)
                    def _far():
                        def fix(k, c):
                            sl = pl.ds(k * 16, 16)
                            own = iota16 + (off + k * 16)
                            iv = idx_v[sl]
                            idx2_v[sl] = jnp.where(iv < S_MIR, own, iv)
                            return c

                        lax.fori_loop(0, NV, fix, 0)
                        g1 = pltpu.async_copy(ps.at[idx2_v], npar2_v, sem1)
                        g2 = pltpu.async_copy(vs.at[idx2_v], gval2_v, sem2)
                        g1.wait()
                        g2.wait()

                    def merge(k, c):
                        sl = pl.ds(k * 16, 16)
                        iv = idx_v[sl]
                        mlo = iv < S_MIR
                        ivc = iv & (S_MIR - 1)
                        gpv = plsc.load_gather(pmir, [ivc])
                        glv = plsc.load_gather(vmir, [ivc])
                        npar_v[sl] = jnp.where(mlo, gpv, npar2_v[sl])
                        val_v[sl] = val_v[sl] + jnp.where(mlo, glv, gval2_v[sl])
                        return c

                    lax.fori_loop(0, NV, merge, 0)
                    pltpu.sync_copy(npar_v, pd.at[pl.ds(off, SUB)])
                    pltpu.sync_copy(val_v, vd.at[pl.ds(off, SUB)])

                @pl.when(m == 0)
                def _conv():
                    pltpu.sync_copy(idx_v, pd.at[pl.ds(off, SUB)])
                    pltpu.sync_copy(val_v, vd.at[pl.ds(off, SUB)])
                    if pd2 is not None:
                        pltpu.sync_copy(idx_v, pd2.at[pl.ds(off, SUB)])
                        pltpu.sync_copy(val_v, vd2.at[pl.ds(off, SUB)])
                    flags[j] = 1

            return carry

        lax.fori_loop(0, NSUB16, sub, 0)
        plsc.subcore_barrier()

    def hop(h, c):
        @pl.when(h == 0)
        def _h0():
            one_hop(par_in, val_in, parA, valA, parB, valB)

        @pl.when((h > 0) & (h % 2 == 1))
        def _hodd():
            one_hop(parA, valA, parB, valB)

        @pl.when((h > 0) & (h % 2 == 0))
        def _heven():
            one_hop(parB, valB, parA, valA)

        return c

    lax.fori_loop(0, NTAIL, hop, 0)


_tail = functools.partial(
    pl.kernel,
    out_type=(jax.ShapeDtypeStruct((NP,), jnp.int32),
              jax.ShapeDtypeStruct((NP,), jnp.float32),
              jax.ShapeDtypeStruct((NP,), jnp.int32),
              jax.ShapeDtypeStruct((NP,), jnp.float32)),
    mesh=_mesh1,
    compiler_params=pltpu.CompilerParams(needs_layout_passes=False),
    scratch_types=[
        pltpu.VMEM((SUB,), jnp.int32),
        pltpu.VMEM((SUB,), jnp.int32),
        pltpu.VMEM((SUB,), jnp.float32),
        pltpu.VMEM((SUB,), jnp.int32),
        pltpu.VMEM((SUB,), jnp.float32),
        pltpu.VMEM((SUB,), jnp.int32),
        pltpu.VMEM((SUB,), jnp.float32),
        pltpu.VMEM((S_MIR,), jnp.int32),
        pltpu.VMEM((S_MIR,), jnp.float32),
        pltpu.SMEM((NSUB16,), jnp.int32),
        pltpu.SemaphoreType.DMA,
        pltpu.SemaphoreType.DMA,
    ],
)(_tail_body)


def _pix_body(pix_in, val_in, out_hbm, idxA, idxB, gvA, gvB,
              sgA, sgB, swA, swB):
    wid = lax.axis_index("s") * 2 + lax.axis_index("c")
    base = wid * PCHUNK

    def pair(p, carry):
        offA = base + (2 * p) * PSUB
        offB = offA + PSUB
        pltpu.sync_copy(pix_in.at[pl.ds(offA, PSUB)], idxA)
        cA = pltpu.async_copy(val_in.at[idxA], gvA, sgA)
        pltpu.sync_copy(pix_in.at[pl.ds(offB, PSUB)], idxB)
        cA.wait()
        wA = pltpu.async_copy(gvA, out_hbm.at[pl.ds(offA, PSUB)], swA)
        cB = pltpu.async_copy(val_in.at[idxB], gvB, sgB)
        cB.wait()
        wB = pltpu.async_copy(gvB, out_hbm.at[pl.ds(offB, PSUB)], swB)
        wA.wait()
        wB.wait()
        return carry

    lax.fori_loop(0, PNSUB // 2, pair, 0)


_pix_gather = functools.partial(
    pl.kernel,
    out_type=jax.ShapeDtypeStruct((PIX,), jnp.float32),
    mesh=_mesh,
    scratch_types=[
        pltpu.VMEM((PSUB,), jnp.int32),
        pltpu.VMEM((PSUB,), jnp.int32),
        pltpu.VMEM((PSUB,), jnp.float32),
        pltpu.VMEM((PSUB,), jnp.float32),
        pltpu.SemaphoreType.DMA,
        pltpu.SemaphoreType.DMA,
        pltpu.SemaphoreType.DMA,
        pltpu.SemaphoreType.DMA,
    ],
)(_pix_body)


def kernel(maxtree_parent, pixel_map, maxtree_diff, attributes, mean,
           inv_diagonal_cov):
    contrib = _scores(attributes, maxtree_diff, mean, inv_diagonal_cov)

    par = jnp.zeros((NP,), jnp.int32).at[1:N + 1].set(
        maxtree_parent.astype(jnp.int32) + 1)
    val = jnp.zeros((NP,), jnp.float32).at[1:N + 1].set(contrib)

    for _ in range(H_SPLIT):
        par, val = _hop(par, val)
    _pa, _va, par, val = _tail(par, val)

    pix = (pixel_map + 1).astype(jnp.int32)
    out = _pix_gather(pix, val)
    return out.reshape(H, W)
